# baseline (speedup 1.0000x reference)
import functools

import jax
import jax.numpy as jnp
from jax import lax
from jax.experimental import pallas as pl
from jax.experimental.pallas import tpu as pltpu
from jax.experimental.pallas import tpu_sc as plsc

N = 10000
G = 100
NPG = 100
E = 320000
D = 128
H = 128
RH = 128
B = 4
MAXLEN = 30

NC = 2
NS = 16
NW = NC * NS

K = 128
SINGLE_CORE = True
NWE = NS if SINGLE_CORE else NC * NS
CH = 160 if SINGLE_CORE else 80
EPAD = NWE * CH * K
CHD = 80
NPAD = 10240
STRIPE = NPAD // NS
SEQ = B * MAXLEN
SEQP = 128

_mesh = plsc.VectorSubcoreMesh(core_axis_name="c", subcore_axis_name="s",
                               num_cores=NC, num_subcores=NS)


@functools.partial(
    pl.kernel,
    out_type=jax.ShapeDtypeStruct((NC, NPAD), jnp.float32),
    mesh=_mesh,
    scratch_types=[
        pltpu.VMEM((CHD, K), jnp.int32),
        pltpu.VMEM((K,), jnp.float32),
        pltpu.VMEM_SHARED((NPAD,), jnp.float32),
    ],
)
def _sc_degree(dst_hbm, zdeg_hbm, ones_hbm, out_hbm, didxs, ones_v, sdeg):
    cid = lax.axis_index("c")
    sid = lax.axis_index("s")
    w = cid * NS + sid
    base = sid * STRIPE
    pltpu.sync_copy(zdeg_hbm.at[pl.ds(base, STRIPE)], sdeg.at[pl.ds(base, STRIPE)])
    pltpu.sync_copy(ones_hbm, ones_v)
    pltpu.sync_copy(dst_hbm.at[w], didxs)
    plsc.subcore_barrier()

    def body(j, carry):
        pltpu.sync_copy(ones_v, sdeg.at[didxs.at[j]], add=True)
        return carry

    lax.fori_loop(0, CHD, body, 0)
    plsc.subcore_barrier()
    pltpu.sync_copy(sdeg.at[pl.ds(base, STRIPE)], out_hbm.at[cid, pl.ds(base, STRIPE)])


def _make_sc_scatter(with_gather: bool):
    outs = [jax.ShapeDtypeStruct((NPAD, H), jnp.float32) if SINGLE_CORE
            else jax.ShapeDtypeStruct((NC, NPAD, H), jnp.float32)]
    scratch = [
        [pltpu.VMEM((2, K), jnp.int32)] * 4,
        [pltpu.VMEM((K, H), jnp.float32)] * 2,
        pltpu.VMEM_SHARED((NPAD, H), jnp.float32),
        [pltpu.SemaphoreType.DMA] * 4,
        [pltpu.SemaphoreType.DMA] * 2,
        [pltpu.SemaphoreType.DMA] * 2,
    ]
    if with_gather:
        outs += [
            jax.ShapeDtypeStruct((SEQP, H), jnp.float32) if SINGLE_CORE
            else jax.ShapeDtypeStruct((NC, SEQP, H), jnp.float32),
            jax.ShapeDtypeStruct((SEQP, H), jnp.float32),
            jax.ShapeDtypeStruct((SEQP,), jnp.float32),
        ]
        scratch += [
            pltpu.VMEM((SEQP,), jnp.int32),
            pltpu.VMEM((SEQP,), jnp.float32),
        ]

    def body(edges_hbm, u_hbm, zrows_hbm, *rest):
        if with_gather:
            (gid_hbm, dtot_hbm, out_hbm, pg_hbm, ug_hbm, dg_hbm,
             idxb, rowsb, acc, isem, gsem, ssem, gidx, gwords) = rest
        else:
            (out_hbm, idxb, rowsb, acc, isem, gsem, ssem) = rest
        rows0 = rowsb[0]
        cid = lax.axis_index("c")
        sid = lax.axis_index("s")
        w = sid if SINGLE_CORE else cid * NS + sid
        base = sid * STRIPE

        def work():
            pltpu.sync_copy(zrows_hbm.at[pl.ds(base, STRIPE)],
                            acc.at[pl.ds(base, STRIPE)])
            plsc.subcore_barrier()

            pltpu.sync_copy(edges_hbm.at[w, 0], idxb[0])
            pltpu.async_copy(edges_hbm.at[w, 1], idxb[1], isem[1])
            pltpu.async_copy(edges_hbm.at[w, 2], idxb[2], isem[2])
            pltpu.async_copy(u_hbm.at[idxb[0].at[0]], rowsb[0], gsem[0])

            def step(ii, carry):
                for u in range(4):
                    j = 4 * ii + u
                    br = u % 2

                    pltpu.make_async_copy(u_hbm.at[idxb[u].at[0]],
                                          rowsb[br], gsem[br]).wait()
                    sd = pltpu.async_copy(rowsb[br], acc.at[idxb[u].at[1]],
                                          ssem[0], add=True)

                    @pl.when(j + 1 < CH)
                    def _():
                        nb = (u + 1) % 4
                        pltpu.make_async_copy(edges_hbm.at[w, j + 1],
                                              idxb[nb], isem[nb]).wait()
                        pltpu.async_copy(u_hbm.at[idxb[nb].at[0]],
                                         rowsb[1 - br], gsem[1 - br])

                    @pl.when(j + 3 < CH)
                    def _():
                        pb = (u + 3) % 4
                        pltpu.async_copy(edges_hbm.at[w, j + 3],
                                         idxb[pb], isem[pb])

                    sd.wait()
                return carry

            lax.fori_loop(0, CH // 4, step, 0)
            plsc.subcore_barrier()
            out_stripe = (out_hbm.at[pl.ds(base, STRIPE)] if SINGLE_CORE
                          else out_hbm.at[cid, pl.ds(base, STRIPE)])
            pltpu.sync_copy(acc.at[pl.ds(base, STRIPE)], out_stripe)
            if with_gather:
                @pl.when(sid == 0)
                def _():
                    pltpu.sync_copy(gid_hbm, gidx)
                    pltpu.sync_copy(acc.at[gidx], rows0)
                    pltpu.sync_copy(rows0, pg_hbm if SINGLE_CORE
                                    else pg_hbm.at[cid])

                    @pl.when(cid == 0)
                    def _():
                        pltpu.sync_copy(u_hbm.at[gidx], rows0)
                        pltpu.sync_copy(rows0, ug_hbm)
                        pltpu.sync_copy(dtot_hbm.at[gidx], gwords)
                        pltpu.sync_copy(gwords, dg_hbm)

        if SINGLE_CORE:
            @pl.when(cid == 0)
            def _():
                work()
        else:
            work()

    return pl.kernel(body, out_type=outs, mesh=_mesh, scratch_types=scratch)


_sc_scatter = _make_sc_scatter(False)
_sc_scatter_gather = _make_sc_scatter(True)


_RB = 512
_GRID = NPAD // _RB


def _tc_b_body(x_ref, w1_ref, d0_ref, d1_ref, u1_ref, dt_ref):
    dt = d0_ref[...] + d1_ref[...] + 1.0
    dinv = lax.rsqrt(dt)
    u1_ref[...] = jnp.dot(x_ref[...], w1_ref[...],
                          preferred_element_type=jnp.float32) * dinv
    dt_ref[...] = dt


def _tc_b(xp, W1, d0, d1):
    return pl.pallas_call(
        _tc_b_body,
        grid=(_GRID,),
        in_specs=[
            pl.BlockSpec((_RB, D), lambda i: (i, 0)),
            pl.BlockSpec((D, H), lambda i: (0, 0)),
            pl.BlockSpec((_RB, 1), lambda i: (i, 0)),
            pl.BlockSpec((_RB, 1), lambda i: (i, 0)),
        ],
        out_specs=[
            pl.BlockSpec((_RB, H), lambda i: (i, 0)),
            pl.BlockSpec((_RB, 1), lambda i: (i, 0)),
        ],
        out_shape=[
            jax.ShapeDtypeStruct((NPAD, H), jnp.float32),
            jax.ShapeDtypeStruct((NPAD, 1), jnp.float32),
        ],
    )(xp, W1, d0, d1)


def _tc_d_body(u1_ref, p0_ref, p1_ref, dt_ref, b1_ref, w2_ref, u2_ref):
    dinv = lax.rsqrt(dt_ref[...])
    h1 = jnp.maximum(dinv * (p0_ref[...] + p1_ref[...] + u1_ref[...])
                     + b1_ref[...], 0.0)
    u2_ref[...] = jnp.dot(h1, w2_ref[...],
                          preferred_element_type=jnp.float32) * dinv


def _tc_d(u1, p0, p1, dt, b1, W2):
    return pl.pallas_call(
        _tc_d_body,
        grid=(_GRID,),
        in_specs=[
            pl.BlockSpec((_RB, H), lambda i: (i, 0)),
            pl.BlockSpec((_RB, H), lambda i: (i, 0)),
            pl.BlockSpec((_RB, H), lambda i: (i, 0)),
            pl.BlockSpec((_RB, 1), lambda i: (i, 0)),
            pl.BlockSpec((1, H), lambda i: (0, 0)),
            pl.BlockSpec((H, H), lambda i: (0, 0)),
        ],
        out_specs=pl.BlockSpec((_RB, H), lambda i: (i, 0)),
        out_shape=jax.ShapeDtypeStruct((NPAD, H), jnp.float32),
    )(u1, p0, p1, dt, b1, W2)


def _tc_f_body(q0_ref, q1_ref, ug_ref, dg_ref, mask_ref, b2_ref,
               wih_ref, bih_ref, whh_ref, bhh_ref, wp_ref, bp_ref, out_ref):
    dinv = lax.rsqrt(dg_ref[...])
    mask = mask_ref[...]
    seq = jnp.maximum(dinv * (q0_ref[...] + q1_ref[...] + ug_ref[...])
                      + b2_ref[...], 0.0) * mask
    dn = (((1,), (1,)), ((), ()))
    gi_all = lax.dot_general(seq, wih_ref[...], dn,
                             preferred_element_type=jnp.float32) + bih_ref[...]
    whh = whh_ref[...]
    bhh = bhh_ref[...]
    h = jnp.zeros((B, RH), jnp.float32)
    outs = []
    for t in range(MAXLEN):
        gi = gi_all[t * B:(t + 1) * B, :]
        gh = lax.dot_general(h, whh, dn,
                             preferred_element_type=jnp.float32) + bhh
        r = jax.nn.sigmoid(gi[:, :RH] + gh[:, :RH])
        z = jax.nn.sigmoid(gi[:, RH:2 * RH] + gh[:, RH:2 * RH])
        n = jnp.tanh(gi[:, 2 * RH:] + r * gh[:, 2 * RH:])
        h = (1.0 - z) * n + z * h
        outs.append(h)
    outs.append(jnp.zeros((SEQP - SEQ, RH), jnp.float32))
    hs = jnp.concatenate(outs, axis=0) * mask
    out_ref[...] = jnp.dot(hs, wp_ref[...],
                           preferred_element_type=jnp.float32) + bp_ref[...]


def _tc_f(q0, q1, ug, dg, maskcol, b2, W_ih, b_ih, W_hh, b_hh, Wp, bp):
    return pl.pallas_call(
        _tc_f_body,
        out_shape=jax.ShapeDtypeStruct((SEQP, 2), jnp.float32),
    )(q0, q1, ug, dg, maskcol, b2, W_ih, b_ih, W_hh, b_hh, Wp, bp)


def kernel(x, edge_index, batch_vec, runner_idx, lengths,
           W1, b1, W2, b2, W_ih, W_hh, b_ih, b_hh, Wp, bp):
    f32 = jnp.float32
    i32 = jnp.int32

    pad = EPAD - E
    srcp = jnp.concatenate([edge_index[0], jnp.zeros((pad,), i32)])
    dstp = jnp.concatenate([edge_index[1], jnp.full((pad,), N, i32)])
    srcp = srcp.reshape(NWE, CH, K)
    dstp = dstp.reshape(NWE, CH, K)
    edges = jnp.stack([srcp, dstp], axis=2)
    padd = NW * CHD * K - E
    dstp_deg = jnp.concatenate([edge_index[1], jnp.full((padd,), N, i32)])
    dstp_deg = dstp_deg.reshape(NW, CHD, K)

    xp = jnp.concatenate([x, jnp.zeros((NPAD - N, D), f32)], axis=0)
    zrows = jnp.zeros((NPAD, H), f32)
    zdeg = jnp.zeros((NPAD,), f32)
    ones_k = jnp.ones((K,), f32)

    starts = jnp.concatenate([jnp.zeros((1,), lengths.dtype),
                              jnp.cumsum(lengths)[:-1]])
    t = jnp.arange(MAXLEN, dtype=i32)
    cidx = jnp.clip(starts[:, None].astype(i32) + t[None, :], 0, G - 1)
    ctm = cidx.T.reshape(-1)
    gid = ctm * NPG + runner_idx[ctm]
    gidp = jnp.concatenate([gid, jnp.zeros((SEQP - SEQ,), i32)])
    maskcol = (t[None, :] < lengths[:, None]).astype(f32).T.reshape(SEQ, 1)
    maskcol = jnp.concatenate([maskcol, jnp.zeros((SEQP - SEQ, 1), f32)])

    deg = _sc_degree(dstp_deg, zdeg, ones_k)
    d0 = deg[0].reshape(NPAD, 1)
    d1 = deg[1].reshape(NPAD, 1)

    u1, dt = _tc_b(xp, W1, d0, d1)

    (p1,) = _sc_scatter(edges, u1, zrows)
    pa, pb = (p1, zrows) if SINGLE_CORE else (p1[0], p1[1])
    u2 = _tc_d(u1, pa, pb, dt, b1.reshape(1, H), W2)

    _, pg, ug, dg = _sc_scatter_gather(edges, u2, zrows, gidp,
                                       dt.reshape(NPAD))
    qa, qb = (pg, zrows[:SEQP]) if SINGLE_CORE else (pg[0], pg[1])

    out = _tc_f(qa, qb, ug, dg.reshape(SEQP, 1), maskcol,
                b2.reshape(1, H), W_ih, b_ih.reshape(1, 3 * RH),
                W_hh, b_hh.reshape(1, 3 * RH), Wp, bp.reshape(1, 2))

    return out[:SEQ].reshape(MAXLEN, B, 2).transpose(1, 0, 2)


# speedup vs baseline: 1.0431x; 1.0431x over previous
import functools

import jax
import jax.numpy as jnp
from jax import lax
from jax.experimental import pallas as pl
from jax.experimental.pallas import tpu as pltpu
from jax.experimental.pallas import tpu_sc as plsc

N = 10000
G = 100
NPG = 100
E = 320000
D = 128
H = 128
RH = 128
B = 4
MAXLEN = 30

NC = 2
NS = 16
NW = NC * NS

K = 128
SINGLE_CORE = True
NWE = NS if SINGLE_CORE else NC * NS
CH = 160 if SINGLE_CORE else 80
EPAD = NWE * CH * K
CHD = 80
NPAD = 10240
STRIPE = NPAD // NS
SEQ = B * MAXLEN
SEQP = 128

_mesh = plsc.VectorSubcoreMesh(core_axis_name="c", subcore_axis_name="s",
                               num_cores=NC, num_subcores=NS)


@functools.partial(
    pl.kernel,
    out_type=jax.ShapeDtypeStruct((NC, NPAD), jnp.float32),
    mesh=_mesh,
    scratch_types=[
        pltpu.VMEM((CHD, K), jnp.int32),
        pltpu.VMEM((K,), jnp.float32),
        pltpu.VMEM_SHARED((NPAD,), jnp.float32),
    ],
)
def _sc_degree(dst_hbm, zdeg_hbm, ones_hbm, out_hbm, didxs, ones_v, sdeg):
    cid = lax.axis_index("c")
    sid = lax.axis_index("s")
    w = cid * NS + sid
    base = sid * STRIPE
    pltpu.sync_copy(zdeg_hbm.at[pl.ds(base, STRIPE)], sdeg.at[pl.ds(base, STRIPE)])
    pltpu.sync_copy(ones_hbm, ones_v)
    pltpu.sync_copy(dst_hbm.at[w], didxs)
    plsc.subcore_barrier()

    def body(j, carry):
        pltpu.sync_copy(ones_v, sdeg.at[didxs.at[j]], add=True)
        return carry

    lax.fori_loop(0, CHD, body, 0)
    plsc.subcore_barrier()
    pltpu.sync_copy(sdeg.at[pl.ds(base, STRIPE)], out_hbm.at[cid, pl.ds(base, STRIPE)])


def _make_sc_scatter(with_gather: bool):
    outs = [jax.ShapeDtypeStruct((NPAD, H), jnp.float32) if SINGLE_CORE
            else jax.ShapeDtypeStruct((NC, NPAD, H), jnp.float32)]
    scratch = [
        [pltpu.VMEM((2, K), jnp.int32)] * 4,
        [pltpu.VMEM((K, H), jnp.float32)] * 2,
        pltpu.VMEM_SHARED((NPAD, H), jnp.float32),
        [pltpu.SemaphoreType.DMA] * 4,
        [pltpu.SemaphoreType.DMA] * 2,
        [pltpu.SemaphoreType.DMA] * 2,
    ]
    if with_gather:
        outs += [
            jax.ShapeDtypeStruct((SEQP, H), jnp.float32) if SINGLE_CORE
            else jax.ShapeDtypeStruct((NC, SEQP, H), jnp.float32),
            jax.ShapeDtypeStruct((SEQP, H), jnp.float32),
            jax.ShapeDtypeStruct((SEQP,), jnp.float32),
        ]
        scratch += [
            pltpu.VMEM((SEQP,), jnp.int32),
            pltpu.VMEM((SEQP,), jnp.float32),
        ]

    def body(edges_hbm, u_hbm, zrows_hbm, *rest):
        if with_gather:
            (gid_hbm, dtot_hbm, out_hbm, pg_hbm, ug_hbm, dg_hbm,
             idxb, rowsb, acc, isem, gsem, ssem, gidx, gwords) = rest
        else:
            (out_hbm, idxb, rowsb, acc, isem, gsem, ssem) = rest
        rows0 = rowsb[0]
        cid = lax.axis_index("c")
        sid = lax.axis_index("s")
        w = sid if SINGLE_CORE else cid * NS + sid
        base = sid * STRIPE

        def work():
            pltpu.sync_copy(zrows_hbm.at[pl.ds(base, STRIPE)],
                            acc.at[pl.ds(base, STRIPE)])
            plsc.subcore_barrier()

            pltpu.sync_copy(edges_hbm.at[w, 0], idxb[0])
            pltpu.async_copy(edges_hbm.at[w, 1], idxb[1], isem[1])
            pltpu.async_copy(u_hbm.at[idxb[0].at[0]], rowsb[0], gsem[0])

            def step(ii, carry):
                for u in range(4):
                    j = 4 * ii + u
                    br = u % 2

                    @pl.when(j + 1 < CH)
                    def _():
                        nb = (u + 1) % 4
                        pltpu.make_async_copy(edges_hbm.at[w, j + 1],
                                              idxb[nb], isem[nb]).wait()
                        pltpu.async_copy(u_hbm.at[idxb[nb].at[0]],
                                         rowsb[1 - br], gsem[1 - br])

                    pltpu.make_async_copy(u_hbm.at[idxb[u].at[0]],
                                          rowsb[br], gsem[br]).wait()
                    pltpu.sync_copy(rowsb[br], acc.at[idxb[u].at[1]], add=True)

                    @pl.when(j + 2 < CH)
                    def _():
                        pb = (u + 2) % 4
                        pltpu.async_copy(edges_hbm.at[w, j + 2],
                                         idxb[pb], isem[pb])
                return carry

            lax.fori_loop(0, CH // 4, step, 0)
            plsc.subcore_barrier()
            out_stripe = (out_hbm.at[pl.ds(base, STRIPE)] if SINGLE_CORE
                          else out_hbm.at[cid, pl.ds(base, STRIPE)])
            pltpu.sync_copy(acc.at[pl.ds(base, STRIPE)], out_stripe)
            if with_gather:
                @pl.when(sid == 0)
                def _():
                    pltpu.sync_copy(gid_hbm, gidx)
                    pltpu.sync_copy(acc.at[gidx], rows0)
                    pltpu.sync_copy(rows0, pg_hbm if SINGLE_CORE
                                    else pg_hbm.at[cid])

                    @pl.when(cid == 0)
                    def _():
                        pltpu.sync_copy(u_hbm.at[gidx], rows0)
                        pltpu.sync_copy(rows0, ug_hbm)
                        pltpu.sync_copy(dtot_hbm.at[gidx], gwords)
                        pltpu.sync_copy(gwords, dg_hbm)

        if SINGLE_CORE:
            @pl.when(cid == 0)
            def _():
                work()
        else:
            work()

    return pl.kernel(body, out_type=outs, mesh=_mesh, scratch_types=scratch)


_sc_scatter = _make_sc_scatter(False)
_sc_scatter_gather = _make_sc_scatter(True)


_RB = 512
_GRID = NPAD // _RB


def _tc_b_body(x_ref, w1_ref, d0_ref, d1_ref, u1_ref, dt_ref):
    dt = d0_ref[...] + d1_ref[...] + 1.0
    dinv = lax.rsqrt(dt)
    u1_ref[...] = jnp.dot(x_ref[...], w1_ref[...],
                          preferred_element_type=jnp.float32) * dinv
    dt_ref[...] = dt


def _tc_b(xp, W1, d0, d1):
    return pl.pallas_call(
        _tc_b_body,
        grid=(_GRID,),
        in_specs=[
            pl.BlockSpec((_RB, D), lambda i: (i, 0)),
            pl.BlockSpec((D, H), lambda i: (0, 0)),
            pl.BlockSpec((_RB, 1), lambda i: (i, 0)),
            pl.BlockSpec((_RB, 1), lambda i: (i, 0)),
        ],
        out_specs=[
            pl.BlockSpec((_RB, H), lambda i: (i, 0)),
            pl.BlockSpec((_RB, 1), lambda i: (i, 0)),
        ],
        out_shape=[
            jax.ShapeDtypeStruct((NPAD, H), jnp.float32),
            jax.ShapeDtypeStruct((NPAD, 1), jnp.float32),
        ],
    )(xp, W1, d0, d1)


def _tc_d_body(u1_ref, p0_ref, p1_ref, dt_ref, b1_ref, w2_ref, u2_ref):
    dinv = lax.rsqrt(dt_ref[...])
    h1 = jnp.maximum(dinv * (p0_ref[...] + p1_ref[...] + u1_ref[...])
                     + b1_ref[...], 0.0)
    u2_ref[...] = jnp.dot(h1, w2_ref[...],
                          preferred_element_type=jnp.float32) * dinv


def _tc_d(u1, p0, p1, dt, b1, W2):
    return pl.pallas_call(
        _tc_d_body,
        grid=(_GRID,),
        in_specs=[
            pl.BlockSpec((_RB, H), lambda i: (i, 0)),
            pl.BlockSpec((_RB, H), lambda i: (i, 0)),
            pl.BlockSpec((_RB, H), lambda i: (i, 0)),
            pl.BlockSpec((_RB, 1), lambda i: (i, 0)),
            pl.BlockSpec((1, H), lambda i: (0, 0)),
            pl.BlockSpec((H, H), lambda i: (0, 0)),
        ],
        out_specs=pl.BlockSpec((_RB, H), lambda i: (i, 0)),
        out_shape=jax.ShapeDtypeStruct((NPAD, H), jnp.float32),
    )(u1, p0, p1, dt, b1, W2)


def _tc_f_body(q0_ref, q1_ref, ug_ref, dg_ref, mask_ref, b2_ref,
               wih_ref, bih_ref, whh_ref, bhh_ref, wp_ref, bp_ref, out_ref):
    dinv = lax.rsqrt(dg_ref[...])
    mask = mask_ref[...]
    seq = jnp.maximum(dinv * (q0_ref[...] + q1_ref[...] + ug_ref[...])
                      + b2_ref[...], 0.0) * mask
    dn = (((1,), (1,)), ((), ()))
    gi_all = lax.dot_general(seq, wih_ref[...], dn,
                             preferred_element_type=jnp.float32) + bih_ref[...]
    whh = whh_ref[...]
    bhh = bhh_ref[...]
    h = jnp.zeros((B, RH), jnp.float32)
    outs = []
    for t in range(MAXLEN):
        gi = gi_all[t * B:(t + 1) * B, :]
        gh = lax.dot_general(h, whh, dn,
                             preferred_element_type=jnp.float32) + bhh
        r = jax.nn.sigmoid(gi[:, :RH] + gh[:, :RH])
        z = jax.nn.sigmoid(gi[:, RH:2 * RH] + gh[:, RH:2 * RH])
        n = jnp.tanh(gi[:, 2 * RH:] + r * gh[:, 2 * RH:])
        h = (1.0 - z) * n + z * h
        outs.append(h)
    outs.append(jnp.zeros((SEQP - SEQ, RH), jnp.float32))
    hs = jnp.concatenate(outs, axis=0) * mask
    out_ref[...] = jnp.dot(hs, wp_ref[...],
                           preferred_element_type=jnp.float32) + bp_ref[...]


def _tc_f(q0, q1, ug, dg, maskcol, b2, W_ih, b_ih, W_hh, b_hh, Wp, bp):
    return pl.pallas_call(
        _tc_f_body,
        out_shape=jax.ShapeDtypeStruct((SEQP, 2), jnp.float32),
    )(q0, q1, ug, dg, maskcol, b2, W_ih, b_ih, W_hh, b_hh, Wp, bp)


def kernel(x, edge_index, batch_vec, runner_idx, lengths,
           W1, b1, W2, b2, W_ih, W_hh, b_ih, b_hh, Wp, bp):
    f32 = jnp.float32
    i32 = jnp.int32

    pad = EPAD - E
    srcp = jnp.concatenate([edge_index[0], jnp.zeros((pad,), i32)])
    dstp = jnp.concatenate([edge_index[1], jnp.full((pad,), N, i32)])
    srcp = srcp.reshape(NWE, CH, K)
    dstp = dstp.reshape(NWE, CH, K)
    edges = jnp.stack([srcp, dstp], axis=2)
    padd = NW * CHD * K - E
    dstp_deg = jnp.concatenate([edge_index[1], jnp.full((padd,), N, i32)])
    dstp_deg = dstp_deg.reshape(NW, CHD, K)

    xp = jnp.concatenate([x, jnp.zeros((NPAD - N, D), f32)], axis=0)
    zrows = jnp.zeros((NPAD, H), f32)
    zdeg = jnp.zeros((NPAD,), f32)
    ones_k = jnp.ones((K,), f32)

    starts = jnp.concatenate([jnp.zeros((1,), lengths.dtype),
                              jnp.cumsum(lengths)[:-1]])
    t = jnp.arange(MAXLEN, dtype=i32)
    cidx = jnp.clip(starts[:, None].astype(i32) + t[None, :], 0, G - 1)
    ctm = cidx.T.reshape(-1)
    gid = ctm * NPG + runner_idx[ctm]
    gidp = jnp.concatenate([gid, jnp.zeros((SEQP - SEQ,), i32)])
    maskcol = (t[None, :] < lengths[:, None]).astype(f32).T.reshape(SEQ, 1)
    maskcol = jnp.concatenate([maskcol, jnp.zeros((SEQP - SEQ, 1), f32)])

    deg = _sc_degree(dstp_deg, zdeg, ones_k)
    d0 = deg[0].reshape(NPAD, 1)
    d1 = deg[1].reshape(NPAD, 1)

    u1, dt = _tc_b(xp, W1, d0, d1)

    (p1,) = _sc_scatter(edges, u1, zrows)
    pa, pb = (p1, zrows) if SINGLE_CORE else (p1[0], p1[1])
    u2 = _tc_d(u1, pa, pb, dt, b1.reshape(1, H), W2)

    _, pg, ug, dg = _sc_scatter_gather(edges, u2, zrows, gidp,
                                       dt.reshape(NPAD))
    qa, qb = (pg, zrows[:SEQP]) if SINGLE_CORE else (pg[0], pg[1])

    out = _tc_f(qa, qb, ug, dg.reshape(SEQP, 1), maskcol,
                b2.reshape(1, H), W_ih, b_ih.reshape(1, 3 * RH),
                W_hh, b_hh.reshape(1, 3 * RH), Wp, bp.reshape(1, 2))

    return out[:SEQ].reshape(MAXLEN, B, 2).transpose(1, 0, 2)


# speedup vs baseline: 1.8151x; 1.7400x over previous
import functools

import jax
import jax.numpy as jnp
from jax import lax
from jax.experimental import pallas as pl
from jax.experimental.pallas import tpu as pltpu
from jax.experimental.pallas import tpu_sc as plsc

N = 10000
G = 100
NPG = 100
E = 320000
D = 128
H = 128
RH = 128
B = 4
MAXLEN = 30

NC = 2
NS = 16
NW = NC * NS

K = 128
CH = 158
EPAD = NS * CH * K
CHD = 80
NPAD = 10240
STRIPE = NPAD // NS
SEQ = B * MAXLEN
SEQP = 128

_mesh = plsc.VectorSubcoreMesh(core_axis_name="c", subcore_axis_name="s",
                               num_cores=NC, num_subcores=NS)


@functools.partial(
    pl.kernel,
    out_type=jax.ShapeDtypeStruct((NC, NPAD), jnp.float32),
    mesh=_mesh,
    scratch_types=[
        pltpu.VMEM((CHD, K), jnp.int32),
        pltpu.VMEM((K,), jnp.float32),
        pltpu.VMEM_SHARED((NPAD,), jnp.float32),
    ],
)
def _sc_degree(dst_hbm, zdeg_hbm, ones_hbm, out_hbm, didxs, ones_v, sdeg):
    cid = lax.axis_index("c")
    sid = lax.axis_index("s")
    w = cid * NS + sid
    base = sid * STRIPE
    pltpu.sync_copy(zdeg_hbm.at[pl.ds(base, STRIPE)], sdeg.at[pl.ds(base, STRIPE)])
    pltpu.sync_copy(ones_hbm, ones_v)
    pltpu.sync_copy(dst_hbm.at[w], didxs)
    plsc.subcore_barrier()

    def body(j, carry):
        pltpu.sync_copy(ones_v, sdeg.at[didxs.at[j]], add=True)
        return carry

    lax.fori_loop(0, CHD, body, 0)
    plsc.subcore_barrier()
    pltpu.sync_copy(sdeg.at[pl.ds(base, STRIPE)], out_hbm.at[cid, pl.ds(base, STRIPE)])


def _make_sc_scatter(with_gather: bool):
    outs = [jax.ShapeDtypeStruct((NPAD, H), jnp.float32)]
    scratch = [
        [pltpu.VMEM((2, K), jnp.int32)] * 2,
        [pltpu.VMEM((K, H), jnp.float32)] * 2,
        pltpu.VMEM_SHARED((NPAD, H), jnp.float32),
        [pltpu.SemaphoreType.DMA] * 2,
        [pltpu.SemaphoreType.DMA] * 2,
    ]
    if with_gather:
        outs += [
            jax.ShapeDtypeStruct((SEQP, H), jnp.float32),
            jax.ShapeDtypeStruct((SEQP, H), jnp.float32),
            jax.ShapeDtypeStruct((SEQP,), jnp.float32),
        ]
        scratch += [
            pltpu.VMEM((SEQP,), jnp.int32),
            pltpu.VMEM((SEQP,), jnp.float32),
        ]

    def body(edges_hbm, u_hbm, zrows_hbm, *rest):
        if with_gather:
            (gid_hbm, dtot_hbm, out_hbm, pg_hbm, ug_hbm, dg_hbm,
             idxb, rowsb, acc, isem, gsem, gidx, gwords) = rest
        else:
            (out_hbm, idxb, rowsb, acc, isem, gsem) = rest
        cid = lax.axis_index("c")
        sid = lax.axis_index("s")
        base = sid * STRIPE

        @pl.when(cid == 0)
        def _():
            w = sid
            pltpu.sync_copy(zrows_hbm.at[pl.ds(base, STRIPE)],
                            acc.at[pl.ds(base, STRIPE)])
            plsc.subcore_barrier()

            pltpu.sync_copy(edges_hbm.at[w, 0], idxb[0])
            pltpu.async_copy(edges_hbm.at[w, 1], idxb[1], isem[1])
            pltpu.async_copy(u_hbm.at[idxb[0].at[0]], rowsb[0], gsem[0])

            def step(jj, carry):
                for b in range(2):
                    j = 2 * jj + b

                    @pl.when(j + 1 < CH)
                    def _():
                        pltpu.make_async_copy(edges_hbm.at[w, j + 1],
                                              idxb[1 - b], isem[1 - b]).wait()
                        pltpu.async_copy(u_hbm.at[idxb[1 - b].at[0]],
                                         rowsb[1 - b], gsem[1 - b])

                    pltpu.make_async_copy(u_hbm.at[idxb[b].at[0]],
                                          rowsb[b], gsem[b]).wait()
                    pltpu.sync_copy(rowsb[b], acc.at[idxb[b].at[1]], add=True)

                    @pl.when(j + 2 < CH)
                    def _():
                        pltpu.async_copy(edges_hbm.at[w, j + 2],
                                         idxb[b], isem[b])
                return carry

            lax.fori_loop(0, CH // 2, step, 0)
            plsc.subcore_barrier()
            pltpu.sync_copy(acc.at[pl.ds(base, STRIPE)],
                            out_hbm.at[pl.ds(base, STRIPE)])
            if with_gather:
                @pl.when(sid == 0)
                def _():
                    pltpu.sync_copy(gid_hbm, gidx)
                    pltpu.sync_copy(acc.at[gidx], rowsb[0])
                    pltpu.sync_copy(rowsb[0], pg_hbm)
                    pltpu.sync_copy(u_hbm.at[gidx], rowsb[0])
                    pltpu.sync_copy(rowsb[0], ug_hbm)
                    pltpu.sync_copy(dtot_hbm.at[gidx], gwords)
                    pltpu.sync_copy(gwords, dg_hbm)

    return pl.kernel(body, out_type=outs, mesh=_mesh, scratch_types=scratch)


_sc_scatter = _make_sc_scatter(False)
_sc_scatter_gather = _make_sc_scatter(True)


_RB = 512
_GRID = NPAD // _RB


def _tc_b_body(x_ref, w1_ref, d0_ref, d1_ref, u1_ref, dt_ref):
    dt = d0_ref[...] + d1_ref[...] + 1.0
    dinv = lax.rsqrt(dt)
    u1_ref[...] = jnp.dot(x_ref[...], w1_ref[...],
                          preferred_element_type=jnp.float32) * dinv
    dt_ref[...] = dt


def _tc_b(xp, W1, d0, d1):
    return pl.pallas_call(
        _tc_b_body,
        grid=(_GRID,),
        in_specs=[
            pl.BlockSpec((_RB, D), lambda i: (i, 0)),
            pl.BlockSpec((D, H), lambda i: (0, 0)),
            pl.BlockSpec((_RB, 1), lambda i: (i, 0)),
            pl.BlockSpec((_RB, 1), lambda i: (i, 0)),
        ],
        out_specs=[
            pl.BlockSpec((_RB, H), lambda i: (i, 0)),
            pl.BlockSpec((_RB, 1), lambda i: (i, 0)),
        ],
        out_shape=[
            jax.ShapeDtypeStruct((NPAD, H), jnp.float32),
            jax.ShapeDtypeStruct((NPAD, 1), jnp.float32),
        ],
    )(xp, W1, d0, d1)


def _tc_d_body(u1_ref, p1_ref, dt_ref, b1_ref, w2_ref, u2_ref):
    dinv = lax.rsqrt(dt_ref[...])
    h1 = jnp.maximum(dinv * (p1_ref[...] + u1_ref[...]) + b1_ref[...], 0.0)
    u2_ref[...] = jnp.dot(h1, w2_ref[...],
                          preferred_element_type=jnp.float32) * dinv


def _tc_d(u1, p1, dt, b1, W2):
    return pl.pallas_call(
        _tc_d_body,
        grid=(_GRID,),
        in_specs=[
            pl.BlockSpec((_RB, H), lambda i: (i, 0)),
            pl.BlockSpec((_RB, H), lambda i: (i, 0)),
            pl.BlockSpec((_RB, 1), lambda i: (i, 0)),
            pl.BlockSpec((1, H), lambda i: (0, 0)),
            pl.BlockSpec((H, H), lambda i: (0, 0)),
        ],
        out_specs=pl.BlockSpec((_RB, H), lambda i: (i, 0)),
        out_shape=jax.ShapeDtypeStruct((NPAD, H), jnp.float32),
    )(u1, p1, dt, b1, W2)


def _tc_f_body(q0_ref, ug_ref, dg_ref, mask_ref, b2_ref,
               wih_ref, bih_ref, whh_ref, bhh_ref, wp_ref, bp_ref, out_ref):
    dinv = lax.rsqrt(dg_ref[...])
    mask = mask_ref[...]
    seq = jnp.maximum(dinv * (q0_ref[...] + ug_ref[...])
                      + b2_ref[...], 0.0) * mask
    dn = (((1,), (1,)), ((), ()))
    gi_all = lax.dot_general(seq, wih_ref[...], dn,
                             preferred_element_type=jnp.float32) + bih_ref[...]
    whh = whh_ref[...]
    bhh = bhh_ref[...]
    h = jnp.zeros((B, RH), jnp.float32)
    outs = []
    for t in range(MAXLEN):
        gi = gi_all[t * B:(t + 1) * B, :]
        gh = lax.dot_general(h, whh, dn,
                             preferred_element_type=jnp.float32) + bhh
        r = jax.nn.sigmoid(gi[:, :RH] + gh[:, :RH])
        z = jax.nn.sigmoid(gi[:, RH:2 * RH] + gh[:, RH:2 * RH])
        n = jnp.tanh(gi[:, 2 * RH:] + r * gh[:, 2 * RH:])
        h = (1.0 - z) * n + z * h
        outs.append(h)
    outs.append(jnp.zeros((SEQP - SEQ, RH), jnp.float32))
    hs = jnp.concatenate(outs, axis=0) * mask
    out_ref[...] = jnp.dot(hs, wp_ref[...],
                           preferred_element_type=jnp.float32) + bp_ref[...]


def _tc_f(q0, ug, dg, maskcol, b2, W_ih, b_ih, W_hh, b_hh, Wp, bp):
    return pl.pallas_call(
        _tc_f_body,
        out_shape=jax.ShapeDtypeStruct((SEQP, 2), jnp.float32),
    )(q0, ug, dg, maskcol, b2, W_ih, b_ih, W_hh, b_hh, Wp, bp)


def kernel(x, edge_index, batch_vec, runner_idx, lengths,
           W1, b1, W2, b2, W_ih, W_hh, b_ih, b_hh, Wp, bp):
    f32 = jnp.float32
    i32 = jnp.int32

    pad = EPAD - E
    srcp = jnp.concatenate([edge_index[0], jnp.zeros((pad,), i32)])
    dstp = jnp.concatenate([edge_index[1], jnp.full((pad,), N, i32)])
    srcp = srcp.reshape(NS, CH, K)
    dstp = dstp.reshape(NS, CH, K)
    edges = jnp.stack([srcp, dstp], axis=2)
    padd = NW * CHD * K - E
    dstp_deg = jnp.concatenate([edge_index[1], jnp.full((padd,), N, i32)])
    dstp_deg = dstp_deg.reshape(NW, CHD, K)

    xp = jnp.concatenate([x, jnp.zeros((NPAD - N, D), f32)], axis=0)
    zrows = jnp.zeros((NPAD, H), f32)
    zdeg = jnp.zeros((NPAD,), f32)
    ones_k = jnp.ones((K,), f32)

    starts = jnp.concatenate([jnp.zeros((1,), lengths.dtype),
                              jnp.cumsum(lengths)[:-1]])
    t = jnp.arange(MAXLEN, dtype=i32)
    cidx = jnp.clip(starts[:, None].astype(i32) + t[None, :], 0, G - 1)
    ctm = cidx.T.reshape(-1)
    gid = ctm * NPG + runner_idx[ctm]
    gidp = jnp.concatenate([gid, jnp.zeros((SEQP - SEQ,), i32)])
    maskcol = (t[None, :] < lengths[:, None]).astype(f32).T.reshape(SEQ, 1)
    maskcol = jnp.concatenate([maskcol, jnp.zeros((SEQP - SEQ, 1), f32)])

    deg = _sc_degree(dstp_deg, zdeg, ones_k)
    d0 = deg[0].reshape(NPAD, 1)
    d1 = deg[1].reshape(NPAD, 1)

    u1, dt = _tc_b(xp, W1, d0, d1)

    (p1,) = _sc_scatter(edges, u1, zrows)
    u2 = _tc_d(u1, p1, dt, b1.reshape(1, H), W2)

    _, pg, ug, dg = _sc_scatter_gather(edges, u2, zrows, gidp,
                                       dt.reshape(NPAD))

    out = _tc_f(pg, ug, dg.reshape(SEQP, 1), maskcol,
                b2.reshape(1, H), W_ih, b_ih.reshape(1, 3 * RH),
                W_hh, b_hh.reshape(1, 3 * RH), Wp, bp.reshape(1, 2))

    return out[:SEQ].reshape(MAXLEN, B, 2).transpose(1, 0, 2)
